# baseline (speedup 1.0000x reference)
import functools

import jax
import jax.numpy as jnp
from jax import lax
from jax.experimental import pallas as pl
from jax.experimental.pallas import tpu as pltpu
from jax.experimental.pallas import tpu_sc as plsc

NUM_CAT = 100
MOST = 10000
CDIM = 32
B = 16384

L = 16
NC = 2
NS = 16
NW = NC * NS
ROWS_PER_W = B // NW
RCHUNK = 16
NCHUNK = ROWS_PER_W // RCHUNK
FPAD = 104
CHUNK_IDX = RCHUNK * FPAD


def _sc_body(xp_hbm, m_hbm, shift_hbm, e_hbm, out_hbm,
             shiftv, idxv0, idxv1, mv0, mv1, embv0, embv1, outv, sem0, sem1):
    wid = lax.axis_index("s") * NC + lax.axis_index("c")
    lanes = jnp.arange(L, dtype=jnp.int32)
    bufs = ((idxv0, mv0, embv0, sem0), (idxv1, mv1, embv1, sem1))

    pltpu.sync_copy(shift_hbm, shiftv)

    def gather_copies(idxb, embb, semb):
        return [
            pltpu.make_async_copy(
                e_hbm.at[idxb.at[pl.ds(r * FPAD, FPAD)]],
                embb.at[pl.ds(r * FPAD, FPAD)],
                semb,
            )
            for r in range(RCHUNK)
        ]

    def stage_fire(ci, bufi):
        idxb, mb, embb, semb = bufs[bufi]
        rowbase = (wid * ROWS_PER_W + ci * RCHUNK).astype(jnp.int32)
        pltpu.sync_copy(xp_hbm.at[pl.ds(rowbase * FPAD, CHUNK_IDX)], idxb)
        pltpu.sync_copy(m_hbm.at[pl.ds(rowbase * NUM_CAT, RCHUNK * NUM_CAT)], mb)
        for j in range(CHUNK_IDX // L):
            sl = pl.ds(j * L, L)
            idxb[sl] = idxb[sl] + shiftv[sl]
        for cp in gather_copies(idxb, embb, semb):
            cp.start()

    def compute_store(ci, bufi):
        idxb, mb, embb, semb = bufs[bufi]
        rowbase = (wid * ROWS_PER_W + ci * RCHUNK).astype(jnp.int32)
        for cp in gather_copies(idxb, embb, semb):
            cp.wait()

        def reduce_f(f, accs):
            w = plsc.load_gather(mb, [lanes * NUM_CAT + f])
            row_i = lanes * FPAD + f
            return tuple(
                accs[d]
                + w * plsc.load_gather(embb, [row_i, jnp.full((L,), d, jnp.int32)])
                for d in range(CDIM)
            )
        zero = jnp.zeros((L,), jnp.float32)
        accs = lax.fori_loop(0, NUM_CAT, reduce_f, (zero,) * CDIM)

        for d in range(CDIM):
            plsc.store_scatter(outv, [lanes, jnp.full((L,), d, jnp.int32)], accs[d])
        pltpu.sync_copy(outv, out_hbm.at[pl.ds(rowbase, RCHUNK)])

    stage_fire(0, 0)

    def outer(i, carry):
        c2 = i * 2
        for b2 in range(2):
            c = c2 + b2
            nxt = c + 1

            @pl.when(nxt < NCHUNK)
            def _():
                stage_fire(nxt, (b2 + 1) % 2)

            compute_store(c, b2)
        return carry

    lax.fori_loop(0, NCHUNK // 2, outer, 0)


def kernel(x, m, E):
    xp = jnp.pad(x.astype(jnp.int32), ((0, 0), (0, FPAD - NUM_CAT))).reshape(-1)
    m_flat = m.reshape(-1)
    j = jnp.arange(CHUNK_IDX, dtype=jnp.int32) % FPAD
    shift = jnp.where(j < NUM_CAT, j * MOST, 0).astype(jnp.int32)
    mesh = plsc.VectorSubcoreMesh(core_axis_name="c", subcore_axis_name="s")
    run = pl.kernel(
        _sc_body,
        out_type=jax.ShapeDtypeStruct((B, CDIM), jnp.float32),
        mesh=mesh,
        scratch_types=[
            pltpu.VMEM((CHUNK_IDX,), jnp.int32),
            pltpu.VMEM((CHUNK_IDX,), jnp.int32),
            pltpu.VMEM((CHUNK_IDX,), jnp.int32),
            pltpu.VMEM((RCHUNK * NUM_CAT,), jnp.float32),
            pltpu.VMEM((RCHUNK * NUM_CAT,), jnp.float32),
            pltpu.VMEM((CHUNK_IDX, CDIM), jnp.float32),
            pltpu.VMEM((CHUNK_IDX, CDIM), jnp.float32),
            pltpu.VMEM((RCHUNK, CDIM), jnp.float32),
            pltpu.SemaphoreType.DMA,
            pltpu.SemaphoreType.DMA,
        ],
        compiler_params=pltpu.CompilerParams(
            needs_layout_passes=False, use_tc_tiling_on_sc=False
        ),
    )
    return run(xp, m_flat, shift, E)
